# baseline (speedup 1.0000x reference)
import functools

import jax
import jax.numpy as jnp
from jax import lax
from jax.experimental import pallas as pl
from jax.experimental.pallas import tpu as pltpu
from jax.experimental.pallas import tpu_sc as plsc

B = 1024
V = 100000
MARGIN = 0.3
SCALE = 32.0

LOG2E = 1.4426950408889634
LN2 = 0.6931471805599453
K2 = SCALE * LOG2E

BLOCK_V = 4096
NB = (V + BLOCK_V - 1) // BLOCK_V
TAIL = V - (NB - 1) * BLOCK_V
PAD = BLOCK_V - TAIL

GRID_R = 2
RB = B // GRID_R

SC_NC = 2
SC_NS = 16
NW = SC_NC * SC_NS
BPW = B // NW

TROWS = (B * V) // 128


def _sc_gather(table_hbm, rowidx_hbm, out_hbm, idx_v, rows_v, sem):
    wid = lax.axis_index("s") * SC_NC + lax.axis_index("c")
    base = wid * BPW
    pltpu.sync_copy(rowidx_hbm.at[pl.ds(base, BPW)], idx_v)
    pltpu.async_copy(table_hbm.at[idx_v], rows_v, sem).wait()
    pltpu.sync_copy(rows_v, out_hbm.at[pl.ds(base, BPW)])


def _gather_label_rows(cosine, rowidx):
    table = cosine.reshape(TROWS, 128)
    mesh = plsc.VectorSubcoreMesh(core_axis_name="c", subcore_axis_name="s")
    return pl.kernel(
        _sc_gather,
        mesh=mesh,
        out_type=jax.ShapeDtypeStruct((B, 128), jnp.float32),
        scratch_types=[
            pltpu.VMEM((BPW,), jnp.int32),
            pltpu.VMEM((BPW, 128), jnp.float32),
            pltpu.SemaphoreType.DMA,
        ],
    )(table, rowidx)


def _tc_kernel(cos_ref, g128_ref, lane_ref, out_ref, m2_ref, s_ref):
    i = pl.program_id(1)

    @pl.when(i == 0)
    def _init():
        m2_ref[...] = jnp.full((RB, 1), -jnp.inf, jnp.float32)
        s_ref[...] = jnp.zeros((RB, 1), jnp.float32)

    cols = lax.broadcasted_iota(jnp.int32, (RB, BLOCK_V), 1) + i * BLOCK_V
    c = jnp.where(cols < V, cos_ref[...], -1e30)
    bm = jnp.max(c, axis=1, keepdims=True) * K2
    m2p = m2_ref[...]
    m2n = jnp.maximum(m2p, bm)
    e = jnp.exp2(c * K2 - m2n)
    alpha = jnp.exp2(m2p - m2n)
    s_ref[...] = s_ref[...] * alpha + jnp.sum(e, axis=1, keepdims=True)
    m2_ref[...] = m2n

    @pl.when(i == NB - 1)
    def _finish():
        lse2 = m2_ref[...] + jnp.log2(s_ref[...])
        lane = lane_ref[...]
        lj = lax.broadcasted_iota(jnp.int32, (RB, 128), 1)
        cl = jnp.sum(
            jnp.where(lj == lane, g128_ref[...], 0.0), axis=1, keepdims=True
        )
        t = jnp.exp2(cl * K2 - lse2)
        d = 2.0 ** (-MARGIN * SCALE * LOG2E)
        term = jnp.maximum(1.0 - t * (1.0 - d), 1e-37)
        lse2m = lse2 + jnp.log2(term)
        loss = LN2 * lse2m - SCALE * (cl - MARGIN)
        out_ref[0, :, :] = jnp.sum(loss, axis=0, keepdims=True) * (1.0 / B)


@functools.partial(jax.jit, static_argnames=("interpret",))
def _amloss_tc(cosine, g128, lane, interpret=False):
    lane2d = lane.reshape(B, 1)
    out = pl.pallas_call(
        _tc_kernel,
        grid=(GRID_R, NB),
        in_specs=[
            pl.BlockSpec((RB, BLOCK_V), lambda r, i: (r, i)),
            pl.BlockSpec((RB, 128), lambda r, i: (r, 0)),
            pl.BlockSpec((RB, 1), lambda r, i: (r, 0)),
        ],
        out_specs=pl.BlockSpec((1, 1, 1), lambda r, i: (r, 0, 0)),
        out_shape=jax.ShapeDtypeStruct((GRID_R, 1, 1), jnp.float32),
        scratch_shapes=[
            pltpu.VMEM((RB, 1), jnp.float32),
            pltpu.VMEM((RB, 1), jnp.float32),
        ],
        compiler_params=pltpu.CompilerParams(
            dimension_semantics=("parallel", "arbitrary")
        ),
        interpret=interpret,
    )(cosine, g128, lane2d)
    return out[0, 0, 0] + out[1, 0, 0]


@jax.jit
def _amloss(cosine, label):
    lab = label.astype(jnp.int32)
    flat = jnp.arange(B, dtype=jnp.int32) * V + lab
    rowidx = flat >> 7
    lane = flat & 127
    g128 = _gather_label_rows(cosine, rowidx)
    return _amloss_tc(cosine, g128, lane)


def kernel(cosine, label):
    return _amloss(cosine, label)


# speedup vs baseline: 1.8000x; 1.8000x over previous
import functools

import jax
import jax.numpy as jnp
from jax import lax
from jax.experimental import pallas as pl
from jax.experimental.pallas import tpu as pltpu

B = 1024
V = 100000
MARGIN = 0.3
SCALE = 32.0

LOG2E = 1.4426950408889634
LN2 = 0.6931471805599453
K2 = SCALE * LOG2E

RBLK = 8
GRID_R = 2
NI = B // RBLK // GRID_R


TSTART = V - 384
TCUT = TSTART + 128


def _tc_kernel(cos_ref, tail_ref, lab_ref, out_ref, acc_ref):
    i = pl.program_id(1)

    @pl.when(i == 0)
    def _init():
        acc_ref[...] = jnp.zeros((1, 1), jnp.float32)

    x2 = cos_ref[...] * K2
    bm = jnp.max(x2, axis=1, keepdims=True)
    s = jnp.sum(jnp.exp2(x2 - bm), axis=1, keepdims=True)
    lse2 = bm + jnp.log2(s)

    segs = []
    lanes = []
    tlanes = []
    for j in range(RBLK):
        lab = lab_ref[j, 0]
        safe = jnp.minimum(lab, TCUT - 1)
        start = pl.multiple_of((safe >> 7) << 7, 128)
        segs.append(cos_ref[pl.ds(j, 1), pl.ds(start, 128)])
        lanes.append(jnp.where(lab < TCUT, safe & 127, -1).reshape(1, 1))
        tlanes.append(jnp.where(lab < TCUT, -1, lab - TSTART).reshape(1, 1))
    seg = jnp.concatenate(segs, axis=0)
    lane = jnp.concatenate(lanes, axis=0)
    tlane = jnp.concatenate(tlanes, axis=0)
    lj = lax.broadcasted_iota(jnp.int32, (RBLK, 128), 1)
    tj = lax.broadcasted_iota(jnp.int32, (RBLK, 384), 1)
    cl = jnp.sum(
        jnp.where(lj == lane, seg, 0.0), axis=1, keepdims=True
    ) + jnp.sum(
        jnp.where(tj == tlane, tail_ref[...], 0.0), axis=1, keepdims=True
    )

    t = jnp.exp2(cl * K2 - lse2)
    d = 2.0 ** (-MARGIN * SCALE * LOG2E)
    term = jnp.maximum(1.0 - t * (1.0 - d), 1e-37)
    lse2m = lse2 + jnp.log2(term)
    loss = LN2 * lse2m - SCALE * (cl - MARGIN)
    acc_ref[...] += jnp.sum(loss, axis=0, keepdims=True)

    @pl.when(i == NI - 1)
    def _finish():
        out_ref[0, :, :] = acc_ref[...] * (1.0 / B)


@functools.partial(jax.jit, static_argnames=("interpret",))
def _amloss(cosine, label, interpret=False):
    lab2d = label.reshape(B, 1).astype(jnp.int32)
    tail = lax.slice(cosine, (0, TSTART), (B, V))
    out = pl.pallas_call(
        _tc_kernel,
        grid=(GRID_R, NI),
        in_specs=[
            pl.BlockSpec((RBLK, V), lambda r, i: (r * NI + i, 0)),
            pl.BlockSpec((RBLK, 384), lambda r, i: (r * NI + i, 0)),
            pl.BlockSpec(
                (RBLK, 1),
                lambda r, i: (r * NI + i, 0),
                memory_space=pltpu.SMEM,
            ),
        ],
        out_specs=pl.BlockSpec((1, 1, 1), lambda r, i: (r, 0, 0)),
        out_shape=jax.ShapeDtypeStruct((GRID_R, 1, 1), jnp.float32),
        scratch_shapes=[
            pltpu.VMEM((1, 1), jnp.float32),
        ],
        compiler_params=pltpu.CompilerParams(
            dimension_semantics=("parallel", "arbitrary")
        ),
        interpret=interpret,
    )(cosine, tail, lab2d)
    return out[0, 0, 0] + out[1, 0, 0]


def kernel(cosine, label):
    return _amloss(cosine, label)


# speedup vs baseline: 2.0518x; 1.1399x over previous
import functools

import jax
import jax.numpy as jnp
from jax import lax
from jax.experimental import pallas as pl
from jax.experimental.pallas import tpu as pltpu

B = 1024
V = 100000
MARGIN = 0.3
SCALE = 32.0

LOG2E = 1.4426950408889634
LN2 = 0.6931471805599453
K2 = SCALE * LOG2E

RBLK = 16
GRID_R = 1
NI = B // RBLK // GRID_R


TSTART = V - 384
TCUT = TSTART + 128


def _tc_kernel(cos_ref, tail_ref, lab_ref, out_ref, acc_ref):
    i = pl.program_id(1)

    @pl.when(i == 0)
    def _init():
        acc_ref[...] = jnp.zeros((1, 1), jnp.float32)

    x2 = cos_ref[...] * K2
    bm = jnp.max(x2, axis=1, keepdims=True)
    s = jnp.sum(jnp.exp2(x2 - bm), axis=1, keepdims=True)
    lse2 = bm + jnp.log2(s)

    segs = []
    lanes = []
    tlanes = []
    for j in range(RBLK):
        lab = lab_ref[j, 0]
        safe = jnp.minimum(lab, TCUT - 1)
        start = pl.multiple_of((safe >> 7) << 7, 128)
        segs.append(cos_ref[pl.ds(j, 1), pl.ds(start, 128)])
        lanes.append(jnp.where(lab < TCUT, safe & 127, -1).reshape(1, 1))
        tlanes.append(jnp.where(lab < TCUT, -1, lab - TSTART).reshape(1, 1))
    seg = jnp.concatenate(segs, axis=0)
    lane = jnp.concatenate(lanes, axis=0)
    tlane = jnp.concatenate(tlanes, axis=0)
    lj = lax.broadcasted_iota(jnp.int32, (RBLK, 128), 1)
    tj = lax.broadcasted_iota(jnp.int32, (RBLK, 384), 1)
    cl = jnp.sum(
        jnp.where(lj == lane, seg, 0.0), axis=1, keepdims=True
    ) + jnp.sum(
        jnp.where(tj == tlane, tail_ref[...], 0.0), axis=1, keepdims=True
    )

    t = jnp.exp2(cl * K2 - lse2)
    d = 2.0 ** (-MARGIN * SCALE * LOG2E)
    term = jnp.maximum(1.0 - t * (1.0 - d), 1e-37)
    lse2m = lse2 + jnp.log2(term)
    loss = LN2 * lse2m - SCALE * (cl - MARGIN)
    acc_ref[...] += jnp.sum(loss, axis=0, keepdims=True)

    @pl.when(i == NI - 1)
    def _finish():
        out_ref[0, :, :] = acc_ref[...] * (1.0 / B)


@functools.partial(jax.jit, static_argnames=("interpret",))
def _amloss(cosine, label, interpret=False):
    lab2d = label.reshape(B, 1).astype(jnp.int32)
    tail = lax.slice(cosine, (0, TSTART), (B, V))
    out = pl.pallas_call(
        _tc_kernel,
        grid=(GRID_R, NI),
        in_specs=[
            pl.BlockSpec((RBLK, V), lambda r, i: (r * NI + i, 0)),
            pl.BlockSpec((RBLK, 384), lambda r, i: (r * NI + i, 0)),
            pl.BlockSpec(
                (RBLK, 1),
                lambda r, i: (r * NI + i, 0),
                memory_space=pltpu.SMEM,
            ),
        ],
        out_specs=pl.BlockSpec((1, 1, 1), lambda r, i: (r, 0, 0)),
        out_shape=jax.ShapeDtypeStruct((GRID_R, 1, 1), jnp.float32),
        scratch_shapes=[
            pltpu.VMEM((1, 1), jnp.float32),
        ],
        compiler_params=pltpu.CompilerParams(
            dimension_semantics=("parallel", "arbitrary")
        ),
        interpret=interpret,
    )(cosine, tail, lab2d)
    return out[0, 0, 0] + out[1, 0, 0]


def kernel(cosine, label):
    return _amloss(cosine, label)


# speedup vs baseline: 2.0607x; 1.0043x over previous
import functools

import jax
import jax.numpy as jnp
from jax import lax
from jax.experimental import pallas as pl
from jax.experimental.pallas import tpu as pltpu

B = 1024
V = 100000
MARGIN = 0.3
SCALE = 32.0

LOG2E = 1.4426950408889634
LN2 = 0.6931471805599453
K2 = SCALE * LOG2E

RBLK = 16
GRID_R = 1
NI = B // RBLK // GRID_R


TSTART = V - 384
TCUT = TSTART + 128


def _tc_kernel(cos_ref, tail_ref, lab_ref, out_ref, acc_ref):
    i = pl.program_id(1)

    @pl.when(i == 0)
    def _init():
        acc_ref[...] = jnp.zeros((1, 1), jnp.float32)

    x2 = cos_ref[...] * K2
    bm = jnp.max(x2, axis=1, keepdims=True)
    s = jnp.sum(jnp.exp2(x2 - bm), axis=1, keepdims=True)
    lse2 = bm + jnp.log2(s)

    segs = []
    lanes = []
    tlanes = []
    for j in range(RBLK):
        lab = lab_ref[j, 0]
        safe = jnp.minimum(lab, TCUT - 1)
        start = pl.multiple_of((safe >> 7) << 7, 128)
        segs.append(cos_ref[pl.ds(j, 1), pl.ds(start, 128)])
        lanes.append(jnp.where(lab < TCUT, safe & 127, -1).reshape(1, 1))
        tlanes.append(jnp.where(lab < TCUT, -1, lab - TSTART).reshape(1, 1))
    seg = jnp.concatenate(segs, axis=0)
    lane = jnp.concatenate(lanes, axis=0)
    tlane = jnp.concatenate(tlanes, axis=0)
    lj = lax.broadcasted_iota(jnp.int32, (RBLK, 128), 1)
    tj = lax.broadcasted_iota(jnp.int32, (RBLK, 384), 1)
    cl = jnp.sum(
        jnp.where(lj == lane, seg, 0.0), axis=1, keepdims=True
    ) + jnp.sum(
        jnp.where(tj == tlane, tail_ref[...], 0.0), axis=1, keepdims=True
    )

    t = jnp.exp2(cl * K2 - lse2)
    d = 2.0 ** (-MARGIN * SCALE * LOG2E)
    term = jnp.maximum(1.0 - t * (1.0 - d), 1e-37)
    lse2m = lse2 + jnp.log2(term)
    loss = LN2 * lse2m - SCALE * (cl - MARGIN)
    acc_ref[...] += jnp.sum(loss, axis=0, keepdims=True)

    @pl.when(i == NI - 1)
    def _finish():
        out_ref[0, :, :] = acc_ref[...] * (1.0 / B)


@functools.partial(jax.jit, static_argnames=("interpret",))
def _amloss(cosine, label, interpret=False):
    lab2d = label.reshape(B, 1).astype(jnp.int32)
    tail = lax.slice(cosine, (0, TSTART), (B, V))
    out = pl.pallas_call(
        _tc_kernel,
        grid=(GRID_R, NI),
        in_specs=[
            pl.BlockSpec((RBLK, V), lambda r, i: (r * NI + i, 0)),
            pl.BlockSpec((RBLK, 384), lambda r, i: (r * NI + i, 0)),
            pl.BlockSpec(
                (RBLK, 1),
                lambda r, i: (r * NI + i, 0),
                memory_space=pltpu.SMEM,
            ),
        ],
        out_specs=pl.BlockSpec((1, 1, 1), lambda r, i: (r, 0, 0)),
        out_shape=jax.ShapeDtypeStruct((GRID_R, 1, 1), jnp.float32),
        scratch_shapes=[
            pltpu.VMEM((1, 1), jnp.float32),
        ],
        compiler_params=pltpu.CompilerParams(
            dimension_semantics=("parallel", "arbitrary")
        ),
        interpret=interpret,
    )(cosine, tail, lab2d)
    return jnp.sum(out)


def kernel(cosine, label):
    return _amloss(cosine, label)


# speedup vs baseline: 2.1880x; 1.0618x over previous
import functools

import jax
import jax.numpy as jnp
from jax import lax
from jax.experimental import pallas as pl
from jax.experimental.pallas import tpu as pltpu

B = 1024
V = 100000
MARGIN = 0.3
SCALE = 32.0

LOG2E = 1.4426950408889634
LN2 = 0.6931471805599453
K2 = SCALE * LOG2E

RBLK = 32
GRID_R = 1
NI = B // RBLK // GRID_R


TSTART = V - 384
TCUT = TSTART + 128


def _tc_kernel(cos_ref, tail_ref, lab_ref, out_ref, acc_ref):
    i = pl.program_id(1)

    @pl.when(i == 0)
    def _init():
        acc_ref[...] = jnp.zeros((1, 1), jnp.float32)

    x2 = cos_ref[...] * K2
    bm = jnp.max(x2, axis=1, keepdims=True)
    s = jnp.sum(jnp.exp2(x2 - bm), axis=1, keepdims=True)
    lse2 = bm + jnp.log2(s)

    segs = []
    lanes = []
    tlanes = []
    for j in range(RBLK):
        lab = lab_ref[j, 0]
        safe = jnp.minimum(lab, TCUT - 1)
        start = pl.multiple_of((safe >> 7) << 7, 128)
        segs.append(cos_ref[pl.ds(j, 1), pl.ds(start, 128)])
        lanes.append(jnp.where(lab < TCUT, safe & 127, -1).reshape(1, 1))
        tlanes.append(jnp.where(lab < TCUT, -1, lab - TSTART).reshape(1, 1))
    seg = jnp.concatenate(segs, axis=0)
    lane = jnp.concatenate(lanes, axis=0)
    tlane = jnp.concatenate(tlanes, axis=0)
    lj = lax.broadcasted_iota(jnp.int32, (RBLK, 128), 1)
    tj = lax.broadcasted_iota(jnp.int32, (RBLK, 384), 1)
    cl = jnp.sum(
        jnp.where(lj == lane, seg, 0.0), axis=1, keepdims=True
    ) + jnp.sum(
        jnp.where(tj == tlane, tail_ref[...], 0.0), axis=1, keepdims=True
    )

    t = jnp.exp2(cl * K2 - lse2)
    d = 2.0 ** (-MARGIN * SCALE * LOG2E)
    term = jnp.maximum(1.0 - t * (1.0 - d), 1e-37)
    lse2m = lse2 + jnp.log2(term)
    loss = LN2 * lse2m - SCALE * (cl - MARGIN)
    acc_ref[...] += jnp.sum(loss, axis=0, keepdims=True)

    @pl.when(i == NI - 1)
    def _finish():
        out_ref[0, :, :] = acc_ref[...] * (1.0 / B)


@functools.partial(jax.jit, static_argnames=("interpret",))
def _amloss(cosine, label, interpret=False):
    lab2d = label.reshape(B, 1).astype(jnp.int32)
    tail = lax.slice(cosine, (0, TSTART), (B, V))
    out = pl.pallas_call(
        _tc_kernel,
        grid=(GRID_R, NI),
        in_specs=[
            pl.BlockSpec((RBLK, V), lambda r, i: (r * NI + i, 0)),
            pl.BlockSpec((RBLK, 384), lambda r, i: (r * NI + i, 0)),
            pl.BlockSpec(
                (RBLK, 1),
                lambda r, i: (r * NI + i, 0),
                memory_space=pltpu.SMEM,
            ),
        ],
        out_specs=pl.BlockSpec((1, 1, 1), lambda r, i: (r, 0, 0)),
        out_shape=jax.ShapeDtypeStruct((GRID_R, 1, 1), jnp.float32),
        scratch_shapes=[
            pltpu.VMEM((1, 1), jnp.float32),
        ],
        compiler_params=pltpu.CompilerParams(
            dimension_semantics=("parallel", "arbitrary")
        ),
        interpret=interpret,
    )(cosine, tail, lab2d)
    return jnp.sum(out)


def kernel(cosine, label):
    return _amloss(cosine, label)
